# baseline (speedup 1.0000x reference)
import functools

import jax
import jax.numpy as jnp
from jax import lax
from jax.experimental import pallas as pl
from jax.experimental.pallas import tpu as pltpu, tpu_sc as plsc

EMB_DIM = 32
L = 16
NBLK = 7812
TAIL_START = NBLK * 128
OROWS = 250000
CHUNK = 128


def _relayout_body(tblT_hbm, o_hbm, in_v, out_v, isem, osem):
    wid = lax.axis_index("s") * 2 + lax.axis_index("c")
    iota16 = lax.iota(jnp.int32, L)
    dvecs = [(16 * t + iota16) % EMB_DIM for t in range(8)]
    svecs = [(16 * t + iota16) // EMB_DIM for t in range(8)]

    def fire_in(b, slot):
        pltpu.async_copy(tblT_hbm.at[:, pl.ds(b * 128, 128)],
                         in_v.at[slot], isem.at[slot])

    def wait_in(slot):
        pltpu.make_async_copy(tblT_hbm.at[:, pl.ds(0, 128)],
                              in_v.at[slot], isem.at[slot]).wait()

    def fire_out(b, slot):
        pltpu.async_copy(out_v.at[slot],
                         o_hbm.at[pl.ds(b * 32, 32), :], osem.at[slot])

    def wait_out(slot):
        pltpu.make_async_copy(o_hbm.at[pl.ds(0, 32), :],
                              out_v.at[slot], osem.at[slot]).wait()

    fire_in(wid, 0)
    fire_in(wid + 32, 1)

    def step(i2, carry):
        for sidx in range(2):
            i = 2 * i2 + sidx
            b = wid + 32 * i

            @pl.when(b < NBLK)
            def _():
                wait_in(sidx)

                @pl.when(i >= 2)
                def _():
                    wait_out(sidx)

                for j in range(32):
                    for t in range(8):
                        v = plsc.load_gather(in_v.at[sidx],
                                             [dvecs[t], 4 * j + svecs[t]])
                        out_v[sidx, j, pl.ds(16 * t, L)] = v
                fire_out(b, sidx)

                @pl.when(b + 64 < NBLK)
                def _():
                    fire_in(b + 64, sidx)
        return carry

    lax.fori_loop(0, 123, step, 0)
    wait_out(0)
    wait_out(1)


def _dot_body(b_per_w, p1_hbm, p2_hbm, o_hbm, wtail_hbm, out_hbm,
              idx1_v, idx2_v, jrow1_v, jrow2_v, wtail_v, out_v,
              rows1, rows2, sem1, sem2):
    wid = lax.axis_index("s") * 2 + lax.axis_index("c")
    base = wid * b_per_w
    pltpu.sync_copy(p1_hbm.at[pl.ds(base, b_per_w)], idx1_v)
    pltpu.sync_copy(p2_hbm.at[pl.ds(base, b_per_w)], idx2_v)
    pltpu.sync_copy(wtail_hbm, wtail_v)
    iota16 = lax.iota(jnp.int32, L)

    for v in range(b_per_w // L):
        sl = pl.ds(v * L, L)
        jrow1_v[sl] = jnp.minimum(idx1_v[sl] >> 2, OROWS - 1)
        jrow2_v[sl] = jnp.minimum(idx2_v[sl] >> 2, OROWS - 1)

    nchunk = b_per_w // CHUNK

    def fire(c, slot):
        pltpu.async_copy(o_hbm.at[jrow1_v.at[pl.ds(c * CHUNK, CHUNK)]],
                         rows1.at[slot], sem1.at[slot])
        pltpu.async_copy(o_hbm.at[jrow2_v.at[pl.ds(c * CHUNK, CHUNK)]],
                         rows2.at[slot], sem2.at[slot])

    def wait(slot):
        pltpu.make_async_copy(o_hbm.at[pl.ds(0, CHUNK), :],
                              rows1.at[slot], sem1.at[slot]).wait()
        pltpu.make_async_copy(o_hbm.at[pl.ds(0, CHUNK), :],
                              rows2.at[slot], sem2.at[slot]).wait()

    fire(0, 0)
    fire(1, 1)

    def chunk_step(c2, carry):
        for slot in range(2):
            c = 2 * c2 + slot
            wait(slot)
            for g in range(CHUNK // L):
                kb = pl.ds(c * CHUNK + g * L, L)
                p1 = idx1_v[kb]
                p2 = idx2_v[kb]
                s1 = (p1 & 3) * EMB_DIM
                s2 = (p2 & 3) * EMB_DIM
                m1 = p1 >= TAIL_START
                m2 = p2 >= TAIL_START
                jt1 = jnp.minimum(jnp.maximum(p1 - TAIL_START, 0), 63)
                jt2 = jnp.minimum(jnp.maximum(p2 - TAIL_START, 0), 63)
                rloc = g * L + iota16
                acc = jnp.zeros((L,), jnp.float32)
                for d in range(EMB_DIM):
                    a = plsc.load_gather(rows1.at[slot], [rloc, s1 + d])
                    b = plsc.load_gather(rows2.at[slot], [rloc, s2 + d])
                    at = plsc.load_gather(wtail_v, [jt1, jnp.full((L,), d, jnp.int32)])
                    bt = plsc.load_gather(wtail_v, [jt2, jnp.full((L,), d, jnp.int32)])
                    a = jnp.where(m1, at, a)
                    b = jnp.where(m2, bt, b)
                    acc = acc + a * b
                out_v[kb] = 1.0 / (1.0 + jnp.exp(-acc))

            @pl.when(c + 2 < nchunk)
            def _():
                fire(c + 2, slot)
        return carry

    lax.fori_loop(0, nchunk // 2, chunk_step, 0)
    pltpu.sync_copy(out_v, out_hbm.at[pl.ds(base, b_per_w)])


def kernel(product1, product2, embedding_weight):
    batch = product1.shape[0]
    info = plsc.get_sparse_core_info()
    nw = info.num_cores * info.num_subcores
    b_per_w = batch // nw
    mesh = plsc.VectorSubcoreMesh(core_axis_name="c", subcore_axis_name="s")
    params = pltpu.CompilerParams(needs_layout_passes=False,
                                  use_tc_tiling_on_sc=True)

    relayout = pl.kernel(
        _relayout_body,
        out_type=jax.ShapeDtypeStruct((OROWS, 128), jnp.float32),
        mesh=mesh,
        scratch_types=[
            pltpu.VMEM((2, EMB_DIM, 128), jnp.float32),
            pltpu.VMEM((2, 32, 128), jnp.float32),
            pltpu.SemaphoreType.DMA((2,)),
            pltpu.SemaphoreType.DMA((2,)),
        ],
        compiler_params=params,
    )

    dot = pl.kernel(
        functools.partial(_dot_body, b_per_w),
        out_type=jax.ShapeDtypeStruct((batch,), jnp.float32),
        mesh=mesh,
        scratch_types=[
            pltpu.VMEM((b_per_w,), jnp.int32),
            pltpu.VMEM((b_per_w,), jnp.int32),
            pltpu.VMEM((b_per_w,), jnp.int32),
            pltpu.VMEM((b_per_w,), jnp.int32),
            pltpu.VMEM((64, 128), jnp.float32),
            pltpu.VMEM((b_per_w,), jnp.float32),
            pltpu.VMEM((2, CHUNK, 128), jnp.float32),
            pltpu.VMEM((2, CHUNK, 128), jnp.float32),
            pltpu.SemaphoreType.DMA((2,)),
            pltpu.SemaphoreType.DMA((2,)),
        ],
        compiler_params=params,
    )

    tblT = embedding_weight.T
    wtail = jnp.pad(embedding_weight[TAIL_START:], ((0, 0), (0, 128 - EMB_DIM)))
    o = relayout(tblT)
    return dot(product1.astype(jnp.int32), product2.astype(jnp.int32),
               o, wtail)


# speedup vs baseline: 1.0009x; 1.0009x over previous
import functools

import jax
import jax.numpy as jnp
from jax import lax
from jax.experimental import pallas as pl
from jax.experimental.pallas import tpu as pltpu, tpu_sc as plsc

EMB_DIM = 32
L = 16
NBLK = 7812
TAIL_START = NBLK * 128
OROWS = 250000
CHUNK = 128


def _relayout_body(tblT_hbm, o_hbm, in_v, out_v, isem, osem):
    wid = lax.axis_index("s") * 2 + lax.axis_index("c")
    iota16 = lax.iota(jnp.int32, L)
    dvecs = [(16 * t + iota16) % EMB_DIM for t in range(8)]
    svecs = [(16 * t + iota16) // EMB_DIM for t in range(8)]

    def fire_in(b, slot):
        pltpu.async_copy(tblT_hbm.at[:, pl.ds(b * 128, 128)],
                         in_v.at[slot], isem.at[slot])

    def wait_in(slot):
        pltpu.make_async_copy(tblT_hbm.at[:, pl.ds(0, 128)],
                              in_v.at[slot], isem.at[slot]).wait()

    def fire_out(b, slot):
        pltpu.async_copy(out_v.at[slot],
                         o_hbm.at[pl.ds(b * 32, 32), :], osem.at[slot])

    def wait_out(slot):
        pltpu.make_async_copy(o_hbm.at[pl.ds(0, 32), :],
                              out_v.at[slot], osem.at[slot]).wait()

    fire_in(wid, 0)
    fire_in(wid + 32, 1)

    def step(i2, carry):
        for sidx in range(2):
            i = 2 * i2 + sidx
            b = wid + 32 * i

            @pl.when(b < NBLK)
            def _():
                wait_in(sidx)

                @pl.when(i >= 2)
                def _():
                    wait_out(sidx)

                for j in range(32):
                    for t in range(8):
                        v = plsc.load_gather(in_v.at[sidx],
                                             [dvecs[t], 4 * j + svecs[t]])
                        out_v[sidx, j, pl.ds(16 * t, L)] = v
                fire_out(b, sidx)

                @pl.when(b + 64 < NBLK)
                def _():
                    fire_in(b + 64, sidx)
        return carry

    lax.fori_loop(0, 123, step, 0)
    wait_out(0)
    wait_out(1)


def _dot_body(b_per_w, p1_hbm, p2_hbm, o_hbm, wtail_hbm, out_hbm,
              idx1_v, idx2_v, jrow1_v, jrow2_v, wtail_v, out_v,
              rows1, rows2, sem1, sem2):
    wid = lax.axis_index("s") * 2 + lax.axis_index("c")
    base = wid * b_per_w
    pltpu.sync_copy(p1_hbm.at[pl.ds(base, b_per_w)], idx1_v)
    pltpu.sync_copy(p2_hbm.at[pl.ds(base, b_per_w)], idx2_v)
    pltpu.sync_copy(wtail_hbm, wtail_v)
    iota16 = lax.iota(jnp.int32, L)

    for v in range(b_per_w // L):
        sl = pl.ds(v * L, L)
        jrow1_v[sl] = jnp.minimum(idx1_v[sl] >> 2, OROWS - 1)
        jrow2_v[sl] = jnp.minimum(idx2_v[sl] >> 2, OROWS - 1)

    nchunk = b_per_w // CHUNK

    def fire(c, slot):
        pltpu.async_copy(o_hbm.at[jrow1_v.at[pl.ds(c * CHUNK, CHUNK)]],
                         rows1.at[slot], sem1.at[slot])
        pltpu.async_copy(o_hbm.at[jrow2_v.at[pl.ds(c * CHUNK, CHUNK)]],
                         rows2.at[slot], sem2.at[slot])

    def wait(slot):
        pltpu.make_async_copy(o_hbm.at[pl.ds(0, CHUNK), :],
                              rows1.at[slot], sem1.at[slot]).wait()
        pltpu.make_async_copy(o_hbm.at[pl.ds(0, CHUNK), :],
                              rows2.at[slot], sem2.at[slot]).wait()

    fire(0, 0)
    fire(1, 1)

    def chunk_step(c2, carry):
        for slot in range(2):
            c = 2 * c2 + slot
            wait(slot)
            for g in range(CHUNK // L):
                kb = pl.ds(c * CHUNK + g * L, L)
                p1 = idx1_v[kb]
                p2 = idx2_v[kb]
                s1 = (p1 & 3) * EMB_DIM
                s2 = (p2 & 3) * EMB_DIM
                m1 = p1 >= TAIL_START
                m2 = p2 >= TAIL_START
                jt1 = jnp.minimum(jnp.maximum(p1 - TAIL_START, 0), 63)
                jt2 = jnp.minimum(jnp.maximum(p2 - TAIL_START, 0), 63)
                rloc = g * L + iota16
                acc = jnp.zeros((L,), jnp.float32)
                for d in range(EMB_DIM):
                    a = plsc.load_gather(rows1.at[slot], [rloc, s1 + d])
                    b = plsc.load_gather(rows2.at[slot], [rloc, s2 + d])
                    at = plsc.load_gather(wtail_v, [jt1, jnp.full((L,), d, jnp.int32)])
                    bt = plsc.load_gather(wtail_v, [jt2, jnp.full((L,), d, jnp.int32)])
                    a = jnp.where(m1, at, a)
                    b = jnp.where(m2, bt, b)
                    acc = acc + a * b
                out_v[kb] = 1.0 / (1.0 + jnp.exp(-acc))

            @pl.when(c + 2 < nchunk)
            def _():
                fire(c + 2, slot)
        return carry

    lax.fori_loop(0, nchunk // 2, chunk_step, 0)
    pltpu.sync_copy(out_v, out_hbm.at[pl.ds(base, b_per_w)])


def kernel(product1, product2, embedding_weight):
    batch = product1.shape[0]
    info = plsc.get_sparse_core_info()
    nw = info.num_cores * info.num_subcores
    b_per_w = batch // nw
    mesh = plsc.VectorSubcoreMesh(core_axis_name="c", subcore_axis_name="s")
    params = pltpu.CompilerParams(needs_layout_passes=False,
                                  use_tc_tiling_on_sc=True,
                                  disable_bounds_checks=True)

    relayout = pl.kernel(
        _relayout_body,
        out_type=jax.ShapeDtypeStruct((OROWS, 128), jnp.float32),
        mesh=mesh,
        scratch_types=[
            pltpu.VMEM((2, EMB_DIM, 128), jnp.float32),
            pltpu.VMEM((2, 32, 128), jnp.float32),
            pltpu.SemaphoreType.DMA((2,)),
            pltpu.SemaphoreType.DMA((2,)),
        ],
        compiler_params=params,
    )

    dot = pl.kernel(
        functools.partial(_dot_body, b_per_w),
        out_type=jax.ShapeDtypeStruct((batch,), jnp.float32),
        mesh=mesh,
        scratch_types=[
            pltpu.VMEM((b_per_w,), jnp.int32),
            pltpu.VMEM((b_per_w,), jnp.int32),
            pltpu.VMEM((b_per_w,), jnp.int32),
            pltpu.VMEM((b_per_w,), jnp.int32),
            pltpu.VMEM((64, 128), jnp.float32),
            pltpu.VMEM((b_per_w,), jnp.float32),
            pltpu.VMEM((2, CHUNK, 128), jnp.float32),
            pltpu.VMEM((2, CHUNK, 128), jnp.float32),
            pltpu.SemaphoreType.DMA((2,)),
            pltpu.SemaphoreType.DMA((2,)),
        ],
        compiler_params=params,
    )

    tblT = embedding_weight.T
    wtail = jnp.pad(embedding_weight[TAIL_START:], ((0, 0), (0, 128 - EMB_DIM)))
    o = relayout(tblT)
    return dot(product1.astype(jnp.int32), product2.astype(jnp.int32),
               o, wtail)


# speedup vs baseline: 1.7804x; 1.7788x over previous
import functools

import jax
import jax.numpy as jnp
from jax import lax
from jax.experimental import pallas as pl
from jax.experimental.pallas import tpu as pltpu, tpu_sc as plsc

EMB_DIM = 32
L = 16
NBLK = 7812
TAIL_START = NBLK * 128
OROWS = 250000
CHUNK = 128


def _relayout_body(tblT_hbm, o_hbm, in_v, out_v, isem, osem):
    wid = lax.axis_index("s") * 2 + lax.axis_index("c")
    iota16 = lax.iota(jnp.int32, L)
    dvecs = [(16 * t + iota16) % EMB_DIM for t in range(8)]
    svecs = [(16 * t + iota16) // EMB_DIM for t in range(8)]

    def fire_in(b, slot):
        pltpu.async_copy(tblT_hbm.at[:, pl.ds(b * 128, 128)],
                         in_v.at[slot], isem.at[slot])

    def wait_in(slot):
        pltpu.make_async_copy(tblT_hbm.at[:, pl.ds(0, 128)],
                              in_v.at[slot], isem.at[slot]).wait()

    def fire_out(b, slot):
        pltpu.async_copy(out_v.at[slot],
                         o_hbm.at[pl.ds(b * 32, 32), :], osem.at[slot])

    def wait_out(slot):
        pltpu.make_async_copy(o_hbm.at[pl.ds(0, 32), :],
                              out_v.at[slot], osem.at[slot]).wait()

    fire_in(wid, 0)
    fire_in(wid + 32, 1)

    def step(i2, carry):
        for sidx in range(2):
            i = 2 * i2 + sidx
            b = wid + 32 * i

            @pl.when(b < NBLK)
            def _():
                wait_in(sidx)

                @pl.when(i >= 2)
                def _():
                    wait_out(sidx)

                @plsc.parallel_loop(0, 32, unroll=4)
                def _transpose(j):
                    cb = j * 4
                    for t in range(8):
                        v = plsc.load_gather(in_v.at[sidx],
                                             [dvecs[t], cb + svecs[t]])
                        out_v[sidx, j, pl.ds(16 * t, L)] = v
                fire_out(b, sidx)

                @pl.when(b + 64 < NBLK)
                def _():
                    fire_in(b + 64, sidx)
        return carry

    lax.fori_loop(0, 123, step, 0)
    wait_out(0)
    wait_out(1)


def _dot_body(b_per_w, p1_hbm, p2_hbm, o_hbm, wtail_hbm, out_hbm,
              idx1_v, idx2_v, jrow1_v, jrow2_v, wtail_v, out_v,
              rows1, rows2, sem1, sem2):
    wid = lax.axis_index("s") * 2 + lax.axis_index("c")
    base = wid * b_per_w
    pltpu.sync_copy(p1_hbm.at[pl.ds(base, b_per_w)], idx1_v)
    pltpu.sync_copy(p2_hbm.at[pl.ds(base, b_per_w)], idx2_v)
    pltpu.sync_copy(wtail_hbm, wtail_v)
    iota16 = lax.iota(jnp.int32, L)

    for v in range(b_per_w // L):
        sl = pl.ds(v * L, L)
        jrow1_v[sl] = jnp.minimum(idx1_v[sl] >> 2, OROWS - 1)
        jrow2_v[sl] = jnp.minimum(idx2_v[sl] >> 2, OROWS - 1)

    nchunk = b_per_w // CHUNK

    def fire(c, slot):
        pltpu.async_copy(o_hbm.at[jrow1_v.at[pl.ds(c * CHUNK, CHUNK)]],
                         rows1.at[slot], sem1.at[slot])
        pltpu.async_copy(o_hbm.at[jrow2_v.at[pl.ds(c * CHUNK, CHUNK)]],
                         rows2.at[slot], sem2.at[slot])

    def wait(slot):
        pltpu.make_async_copy(o_hbm.at[pl.ds(0, CHUNK), :],
                              rows1.at[slot], sem1.at[slot]).wait()
        pltpu.make_async_copy(o_hbm.at[pl.ds(0, CHUNK), :],
                              rows2.at[slot], sem2.at[slot]).wait()

    fire(0, 0)
    fire(1, 1)

    def chunk_step(c2, carry):
        for slot in range(2):
            c = 2 * c2 + slot
            wait(slot)
            for g in range(CHUNK // L):
                kb = pl.ds(c * CHUNK + g * L, L)
                p1 = idx1_v[kb]
                p2 = idx2_v[kb]
                s1 = (p1 & 3) * EMB_DIM
                s2 = (p2 & 3) * EMB_DIM
                m1 = p1 >= TAIL_START
                m2 = p2 >= TAIL_START
                jt1 = jnp.minimum(jnp.maximum(p1 - TAIL_START, 0), 63)
                jt2 = jnp.minimum(jnp.maximum(p2 - TAIL_START, 0), 63)
                rloc = g * L + iota16
                acc = jnp.zeros((L,), jnp.float32)
                for d in range(EMB_DIM):
                    a = plsc.load_gather(rows1.at[slot], [rloc, s1 + d])
                    b = plsc.load_gather(rows2.at[slot], [rloc, s2 + d])
                    at = plsc.load_gather(wtail_v, [jt1, jnp.full((L,), d, jnp.int32)])
                    bt = plsc.load_gather(wtail_v, [jt2, jnp.full((L,), d, jnp.int32)])
                    a = jnp.where(m1, at, a)
                    b = jnp.where(m2, bt, b)
                    acc = acc + a * b
                out_v[kb] = 1.0 / (1.0 + jnp.exp(-acc))

            @pl.when(c + 2 < nchunk)
            def _():
                fire(c + 2, slot)
        return carry

    lax.fori_loop(0, nchunk // 2, chunk_step, 0)
    pltpu.sync_copy(out_v, out_hbm.at[pl.ds(base, b_per_w)])


def kernel(product1, product2, embedding_weight):
    batch = product1.shape[0]
    info = plsc.get_sparse_core_info()
    nw = info.num_cores * info.num_subcores
    b_per_w = batch // nw
    mesh = plsc.VectorSubcoreMesh(core_axis_name="c", subcore_axis_name="s")
    params = pltpu.CompilerParams(needs_layout_passes=False,
                                  use_tc_tiling_on_sc=True,
                                  disable_bounds_checks=True)

    relayout = pl.kernel(
        _relayout_body,
        out_type=jax.ShapeDtypeStruct((OROWS, 128), jnp.float32),
        mesh=mesh,
        scratch_types=[
            pltpu.VMEM((2, EMB_DIM, 128), jnp.float32),
            pltpu.VMEM((2, 32, 128), jnp.float32),
            pltpu.SemaphoreType.DMA((2,)),
            pltpu.SemaphoreType.DMA((2,)),
        ],
        compiler_params=params,
    )

    dot = pl.kernel(
        functools.partial(_dot_body, b_per_w),
        out_type=jax.ShapeDtypeStruct((batch,), jnp.float32),
        mesh=mesh,
        scratch_types=[
            pltpu.VMEM((b_per_w,), jnp.int32),
            pltpu.VMEM((b_per_w,), jnp.int32),
            pltpu.VMEM((b_per_w,), jnp.int32),
            pltpu.VMEM((b_per_w,), jnp.int32),
            pltpu.VMEM((64, 128), jnp.float32),
            pltpu.VMEM((b_per_w,), jnp.float32),
            pltpu.VMEM((2, CHUNK, 128), jnp.float32),
            pltpu.VMEM((2, CHUNK, 128), jnp.float32),
            pltpu.SemaphoreType.DMA((2,)),
            pltpu.SemaphoreType.DMA((2,)),
        ],
        compiler_params=params,
    )

    tblT = embedding_weight.T
    wtail = jnp.pad(embedding_weight[TAIL_START:], ((0, 0), (0, 128 - EMB_DIM)))
    o = relayout(tblT)
    return dot(product1.astype(jnp.int32), product2.astype(jnp.int32),
               o, wtail)
